# baseline (speedup 1.0000x reference)
import jax
import jax.numpy as jnp
from jax.experimental import pallas as pl
from jax.experimental.pallas import tpu as pltpu


def _conv3x3_flat(y, w_ref):
    y = y.astype(jnp.bfloat16)
    H, W, C = y.shape
    zrow = jnp.zeros((1, W, C), y.dtype)
    yp = jnp.concatenate([zrow, y, zrow], axis=0)
    zcol = jnp.zeros((H + 2, 1, C), y.dtype)
    yl = jnp.concatenate([yp[:, 1:, :], zcol], axis=1)
    yr = jnp.concatenate([zcol, yp[:, : W - 1, :]], axis=1)
    srcs = [yr.reshape((H + 2) * W, C),
            yp.reshape((H + 2) * W, C),
            yl.reshape((H + 2) * W, C)]
    cols = jnp.concatenate(
        [srcs[j][i * W: i * W + H * W, :] for i in range(3) for j in range(3)],
        axis=-1)
    return jnp.dot(cols, w_ref[...], preferred_element_type=jnp.float32)


def _affine(s_ref, sq_ref, prm_ref, count):
    tot = jnp.sum(s_ref[...], axis=(0, 1))
    tot_sq = jnp.sum(sq_ref[...], axis=(0, 1))
    mean = tot / count
    var = jnp.maximum(tot_sq / count - mean * mean, 0.0)
    scale = prm_ref[0] / jnp.sqrt(var + 1e-5)
    shift = prm_ref[1] - mean * scale
    return scale, shift, prm_ref[2]


def _bn_prelu(x, aff):
    scale, shift, alpha = aff
    y = x * scale + shift
    return jnp.where(y >= 0.0, y, alpha * y)


def _stats_out(acc, s_ref, sq_ref):
    s_ref[0] = jnp.sum(acc, axis=0, keepdims=True)
    sq_ref[0] = jnp.sum(acc * acc, axis=0, keepdims=True)


def _convt_body(xd_ref, w_ref, o_ref, s_ref, sq_ref):
    xd = xd_ref[0]
    Hp, Wp, C = xd.shape
    y = xd[1:Hp - 1, 1:Wp - 1, :]
    acc = _conv3x3_flat(y, w_ref)
    o_ref[0] = acc.reshape(Hp - 2, Wp - 2, C)
    _stats_out(acc, s_ref, sq_ref)


def _make_conv0_body(count):
    def body(skc_ref, up_ref, s0_ref, sq0_ref, prm_ref, w_ref,
             o_ref, s_ref, sq_ref):
        u = _bn_prelu(up_ref[0], _affine(s0_ref, sq0_ref, prm_ref, count))
        sk = jnp.transpose(skc_ref[0], (1, 2, 0))
        y = jnp.concatenate([sk, u], axis=-1)
        acc = _conv3x3_flat(y, w_ref)
        H, W, _ = y.shape
        o_ref[0] = acc.reshape(H, W, o_ref.shape[-1])
        _stats_out(acc, s_ref, sq_ref)
    return body


def _make_conv_body(count):
    def body(x_ref, s0_ref, sq0_ref, prm_ref, w_ref, o_ref, s_ref, sq_ref):
        y = _bn_prelu(x_ref[0], _affine(s0_ref, sq0_ref, prm_ref, count))
        acc = _conv3x3_flat(y, w_ref)
        H, W, _ = y.shape
        o_ref[0] = acc.reshape(H, W, o_ref.shape[-1])
        _stats_out(acc, s_ref, sq_ref)
    return body


def _make_resid_body(count):
    def body(y_ref, sy_ref, sqy_ref, prmy_ref, cb_ref, s2_ref, sq2_ref,
             prm2_ref, w_ref, b_ref, o_ref):
        y = _bn_prelu(y_ref[0], _affine(sy_ref, sqy_ref, prmy_ref, count))
        cb = _bn_prelu(cb_ref[0], _affine(s2_ref, sq2_ref, prm2_ref, count))
        H, W, C = y.shape
        n = w_ref.shape[1]
        out = jnp.dot(y.astype(jnp.bfloat16).reshape(H * W, C), w_ref[...],
                      preferred_element_type=jnp.float32)
        out = out.reshape(H, W, n) + b_ref[0] + cb
        o_ref[0] = jnp.transpose(out, (2, 0, 1))
    return body


_PAR = pltpu.CompilerParams(dimension_semantics=("parallel",))


def _conv_stage(body, tensors, specs, out_shapes, out_specs, B):
    return pl.pallas_call(
        body,
        out_shape=out_shapes,
        grid=(B,),
        in_specs=specs,
        out_specs=out_specs,
        compiler_params=_PAR,
    )(*tensors)


def kernel(convT_w, convT_b, bn0_g, bn0_b, prelu0_a, conv0_w, conv0_b,
           bn1_g, bn1_b, prelu1_a, rb_c1_w, rb_c1_b, rb_bn1_g, rb_bn1_b,
           rb_prelu1_a, rb_c2_w, rb_c2_b, rb_bn2_g, rb_bn2_b, rb_prelu2_a,
           rb_c11_w, rb_c11_b, x, skip):
    xh = jnp.transpose(x, (0, 2, 3, 1)).astype(jnp.float32)
    skip = skip.astype(jnp.float32)
    B, H, W, c = xh.shape
    Hs, Ws = skip.shape[2], skip.shape[3]
    n = conv0_w.shape[0]
    count = B * Hs * Ws

    op_h = Hs - ((H - 1) * 2 - 2 + 3)
    op_w = Ws - ((W - 1) * 2 - 2 + 3)
    hd, wd = 2 * H - 1, 2 * W - 1
    xd = jnp.zeros((B, hd, wd, c), jnp.float32).at[:, ::2, ::2, :].set(xh)
    xd = jnp.pad(xd, ((0, 0), (1, 1 + op_h), (1, 1 + op_w), (0, 0)))

    wt = jnp.transpose(jnp.flip(convT_w, (2, 3)), (2, 3, 0, 1)).astype(
        jnp.bfloat16).reshape(9 * c, c)
    w0 = jnp.transpose(conv0_w, (2, 3, 1, 0)).astype(jnp.bfloat16).reshape(
        18 * c, n)
    w1 = jnp.transpose(rb_c1_w, (2, 3, 1, 0)).astype(jnp.bfloat16).reshape(9 * n, n)
    w2 = jnp.transpose(rb_c2_w, (2, 3, 1, 0)).astype(jnp.bfloat16).reshape(9 * n, n)
    w11 = rb_c11_w[:, :, 0, 0].T.astype(jnp.bfloat16)
    b11 = rb_c11_b.reshape(1, -1).astype(jnp.float32)

    def prm(g, b_, alpha):
        a = jnp.broadcast_to(jnp.asarray(alpha, jnp.float32).reshape(()),
                             g.shape)
        return jnp.stack([g.astype(jnp.float32), b_.astype(jnp.float32), a], 0)

    prm0 = prm(bn0_g, bn0_b, prelu0_a)
    prm1 = prm(bn1_g, bn1_b, prelu1_a)
    prm_rb1 = prm(rb_bn1_g, rb_bn1_b, rb_prelu1_a)
    prm_rb2 = prm(rb_bn2_g, rb_bn2_b, rb_prelu2_a)

    full = lambda *dims: pl.BlockSpec(dims, lambda b: (0,) * len(dims))
    perb = lambda *dims: pl.BlockSpec((1,) + dims,
                                      lambda b: (b,) + (0,) * len(dims))
    stat_shapes = lambda ch: (jax.ShapeDtypeStruct((B, 1, ch), jnp.float32),
                              jax.ShapeDtypeStruct((B, 1, ch), jnp.float32))
    sspec = lambda ch: pl.BlockSpec((1, 1, ch), lambda b: (b, 0, 0))

    up, s0, sq0 = _conv_stage(
        _convt_body, (xd, wt),
        [perb(Hs + 2, Ws + 2, c), full(9 * c, c)],
        (jax.ShapeDtypeStruct((B, Hs, Ws, c), jnp.float32),) + stat_shapes(c),
        (perb(Hs, Ws, c), sspec(c), sspec(c)), B)

    y_raw, sy, sqy = _conv_stage(
        _make_conv0_body(count), (skip, up, s0, sq0, prm0, w0),
        [perb(c, Hs, Ws), perb(Hs, Ws, c), full(B, 1, c), full(B, 1, c),
         full(3, c), full(18 * c, n)],
        (jax.ShapeDtypeStruct((B, Hs, Ws, n), jnp.float32),) + stat_shapes(n),
        (perb(Hs, Ws, n), sspec(n), sspec(n)), B)

    cb1, s1, sq1 = _conv_stage(
        _make_conv_body(count), (y_raw, sy, sqy, prm1, w1),
        [perb(Hs, Ws, n), full(B, 1, n), full(B, 1, n), full(3, n),
         full(9 * n, n)],
        (jax.ShapeDtypeStruct((B, Hs, Ws, n), jnp.float32),) + stat_shapes(n),
        (perb(Hs, Ws, n), sspec(n), sspec(n)), B)

    cb2, s2, sq2 = _conv_stage(
        _make_conv_body(count), (cb1, s1, sq1, prm_rb1, w2),
        [perb(Hs, Ws, n), full(B, 1, n), full(B, 1, n), full(3, n),
         full(9 * n, n)],
        (jax.ShapeDtypeStruct((B, Hs, Ws, n), jnp.float32),) + stat_shapes(n),
        (perb(Hs, Ws, n), sspec(n), sspec(n)), B)

    out = _conv_stage(
        _make_resid_body(count),
        (y_raw, sy, sqy, prm1, cb2, s2, sq2, prm_rb2, w11, b11),
        [perb(Hs, Ws, n), full(B, 1, n), full(B, 1, n), full(3, n),
         perb(Hs, Ws, n), full(B, 1, n), full(B, 1, n), full(3, n),
         full(n, n), full(1, n)],
        jax.ShapeDtypeStruct((B, n, Hs, Ws), jnp.float32),
        perb(n, Hs, Ws), B)
    return out


# speedup vs baseline: 1.0376x; 1.0376x over previous
import jax
import jax.numpy as jnp
from jax.experimental import pallas as pl
from jax.experimental.pallas import tpu as pltpu


def _conv3x3_flat(y, w_ref):
    y = y.astype(jnp.bfloat16)
    H, W, C = y.shape
    zrow = jnp.zeros((1, W, C), y.dtype)
    yp = jnp.concatenate([zrow, y, zrow], axis=0)
    zcol = jnp.zeros((H + 2, 1, C), y.dtype)
    yl = jnp.concatenate([yp[:, 1:, :], zcol], axis=1)
    yr = jnp.concatenate([zcol, yp[:, : W - 1, :]], axis=1)
    srcs = [yr.reshape((H + 2) * W, C),
            yp.reshape((H + 2) * W, C),
            yl.reshape((H + 2) * W, C)]
    cols = jnp.concatenate(
        [srcs[j][i * W: i * W + H * W, :] for i in range(3) for j in range(3)],
        axis=-1)
    return jnp.dot(cols, w_ref[...], preferred_element_type=jnp.float32)


def _affine(s_ref, sq_ref, prm_ref, count):
    tot = jnp.sum(s_ref[...], axis=(0, 1))
    tot_sq = jnp.sum(sq_ref[...], axis=(0, 1))
    mean = tot / count
    var = jnp.maximum(tot_sq / count - mean * mean, 0.0)
    scale = prm_ref[0] / jnp.sqrt(var + 1e-5)
    shift = prm_ref[1] - mean * scale
    return scale, shift, prm_ref[2]


def _bn_prelu(x, aff):
    scale, shift, alpha = aff
    y = x * scale + shift
    return jnp.where(y >= 0.0, y, alpha * y)


def _stats_out(acc, s_ref, sq_ref):
    s_ref[0] = jnp.sum(acc, axis=0, keepdims=True)
    sq_ref[0] = jnp.sum(acc * acc, axis=0, keepdims=True)


def _convt_body(xd_ref, w_ref, o_ref, s_ref, sq_ref):
    xd = xd_ref[0]
    Hp, Wp, C = xd.shape
    y = xd[1:Hp - 1, 1:Wp - 1, :]
    acc = _conv3x3_flat(y, w_ref)
    o_ref[0] = acc.reshape(Hp - 2, Wp - 2, C)
    _stats_out(acc, s_ref, sq_ref)


def _make_conv0_body(count):
    def body(skc_ref, up_ref, s0_ref, sq0_ref, prm_ref, w_ref,
             o_ref, s_ref, sq_ref):
        u = _bn_prelu(up_ref[0], _affine(s0_ref, sq0_ref, prm_ref, count))
        y = jnp.concatenate([skc_ref[0], u], axis=-1)
        acc = _conv3x3_flat(y, w_ref)
        H, W, _ = y.shape
        o_ref[0] = acc.reshape(H, W, o_ref.shape[-1])
        _stats_out(acc, s_ref, sq_ref)
    return body


def _make_conv_body(count):
    def body(x_ref, s0_ref, sq0_ref, prm_ref, w_ref, o_ref, s_ref, sq_ref):
        y = _bn_prelu(x_ref[0], _affine(s0_ref, sq0_ref, prm_ref, count))
        acc = _conv3x3_flat(y, w_ref)
        H, W, _ = y.shape
        o_ref[0] = acc.reshape(H, W, o_ref.shape[-1])
        _stats_out(acc, s_ref, sq_ref)
    return body


def _make_resid_body(count):
    def body(y_ref, sy_ref, sqy_ref, prmy_ref, cb_ref, s2_ref, sq2_ref,
             prm2_ref, w_ref, b_ref, o_ref):
        y = _bn_prelu(y_ref[0], _affine(sy_ref, sqy_ref, prmy_ref, count))
        cb = _bn_prelu(cb_ref[0], _affine(s2_ref, sq2_ref, prm2_ref, count))
        H, W, C = y.shape
        n = w_ref.shape[1]
        out = jnp.dot(y.astype(jnp.bfloat16).reshape(H * W, C), w_ref[...],
                      preferred_element_type=jnp.float32)
        out = out.reshape(H, W, n) + b_ref[0] + cb
        o_ref[0] = jnp.transpose(out, (2, 0, 1))
    return body


_PAR = pltpu.CompilerParams(dimension_semantics=("parallel",))


def _conv_stage(body, tensors, specs, out_shapes, out_specs, B):
    return pl.pallas_call(
        body,
        out_shape=out_shapes,
        grid=(B,),
        in_specs=specs,
        out_specs=out_specs,
        compiler_params=_PAR,
    )(*tensors)


def kernel(convT_w, convT_b, bn0_g, bn0_b, prelu0_a, conv0_w, conv0_b,
           bn1_g, bn1_b, prelu1_a, rb_c1_w, rb_c1_b, rb_bn1_g, rb_bn1_b,
           rb_prelu1_a, rb_c2_w, rb_c2_b, rb_bn2_g, rb_bn2_b, rb_prelu2_a,
           rb_c11_w, rb_c11_b, x, skip):
    xh = jnp.transpose(x, (0, 2, 3, 1)).astype(jnp.float32)
    skip = jnp.transpose(skip, (0, 2, 3, 1)).astype(jnp.float32)
    B, H, W, c = xh.shape
    Hs, Ws = skip.shape[1], skip.shape[2]
    n = conv0_w.shape[0]
    count = B * Hs * Ws

    op_h = Hs - ((H - 1) * 2 - 2 + 3)
    op_w = Ws - ((W - 1) * 2 - 2 + 3)
    hd, wd = 2 * H - 1, 2 * W - 1
    xd = jnp.zeros((B, hd, wd, c), jnp.float32).at[:, ::2, ::2, :].set(xh)
    xd = jnp.pad(xd, ((0, 0), (1, 1 + op_h), (1, 1 + op_w), (0, 0)))

    wt = jnp.transpose(jnp.flip(convT_w, (2, 3)), (2, 3, 0, 1)).astype(
        jnp.bfloat16).reshape(9 * c, c)
    w0 = jnp.transpose(conv0_w, (2, 3, 1, 0)).astype(jnp.bfloat16).reshape(
        18 * c, n)
    w1 = jnp.transpose(rb_c1_w, (2, 3, 1, 0)).astype(jnp.bfloat16).reshape(9 * n, n)
    w2 = jnp.transpose(rb_c2_w, (2, 3, 1, 0)).astype(jnp.bfloat16).reshape(9 * n, n)
    w11 = rb_c11_w[:, :, 0, 0].T.astype(jnp.bfloat16)
    b11 = rb_c11_b.reshape(1, -1).astype(jnp.float32)

    def prm(g, b_, alpha):
        a = jnp.broadcast_to(jnp.asarray(alpha, jnp.float32).reshape(()),
                             g.shape)
        return jnp.stack([g.astype(jnp.float32), b_.astype(jnp.float32), a], 0)

    prm0 = prm(bn0_g, bn0_b, prelu0_a)
    prm1 = prm(bn1_g, bn1_b, prelu1_a)
    prm_rb1 = prm(rb_bn1_g, rb_bn1_b, rb_prelu1_a)
    prm_rb2 = prm(rb_bn2_g, rb_bn2_b, rb_prelu2_a)

    full = lambda *dims: pl.BlockSpec(dims, lambda b: (0,) * len(dims))
    perb = lambda *dims: pl.BlockSpec((1,) + dims,
                                      lambda b: (b,) + (0,) * len(dims))
    stat_shapes = lambda ch: (jax.ShapeDtypeStruct((B, 1, ch), jnp.float32),
                              jax.ShapeDtypeStruct((B, 1, ch), jnp.float32))
    sspec = lambda ch: pl.BlockSpec((1, 1, ch), lambda b: (b, 0, 0))

    up, s0, sq0 = _conv_stage(
        _convt_body, (xd, wt),
        [perb(Hs + 2, Ws + 2, c), full(9 * c, c)],
        (jax.ShapeDtypeStruct((B, Hs, Ws, c), jnp.float32),) + stat_shapes(c),
        (perb(Hs, Ws, c), sspec(c), sspec(c)), B)

    y_raw, sy, sqy = _conv_stage(
        _make_conv0_body(count), (skip, up, s0, sq0, prm0, w0),
        [perb(Hs, Ws, c), perb(Hs, Ws, c), full(B, 1, c), full(B, 1, c),
         full(3, c), full(18 * c, n)],
        (jax.ShapeDtypeStruct((B, Hs, Ws, n), jnp.float32),) + stat_shapes(n),
        (perb(Hs, Ws, n), sspec(n), sspec(n)), B)

    cb1, s1, sq1 = _conv_stage(
        _make_conv_body(count), (y_raw, sy, sqy, prm1, w1),
        [perb(Hs, Ws, n), full(B, 1, n), full(B, 1, n), full(3, n),
         full(9 * n, n)],
        (jax.ShapeDtypeStruct((B, Hs, Ws, n), jnp.float32),) + stat_shapes(n),
        (perb(Hs, Ws, n), sspec(n), sspec(n)), B)

    cb2, s2, sq2 = _conv_stage(
        _make_conv_body(count), (cb1, s1, sq1, prm_rb1, w2),
        [perb(Hs, Ws, n), full(B, 1, n), full(B, 1, n), full(3, n),
         full(9 * n, n)],
        (jax.ShapeDtypeStruct((B, Hs, Ws, n), jnp.float32),) + stat_shapes(n),
        (perb(Hs, Ws, n), sspec(n), sspec(n)), B)

    out = _conv_stage(
        _make_resid_body(count),
        (y_raw, sy, sqy, prm1, cb2, s2, sq2, prm_rb2, w11, b11),
        [perb(Hs, Ws, n), full(B, 1, n), full(B, 1, n), full(3, n),
         perb(Hs, Ws, n), full(B, 1, n), full(B, 1, n), full(3, n),
         full(n, n), full(1, n)],
        jax.ShapeDtypeStruct((B, n, Hs, Ws), jnp.float32),
        perb(n, Hs, Ws), B)
    return out


# speedup vs baseline: 1.2593x; 1.2137x over previous
import jax
import jax.numpy as jnp
from jax.experimental import pallas as pl
from jax.experimental.pallas import tpu as pltpu


def _conv3x3_flat(y, w_ref):
    y = y.astype(jnp.bfloat16)
    H, W, C = y.shape
    zrow = jnp.zeros((1, W, C), y.dtype)
    yp = jnp.concatenate([zrow, y, zrow], axis=0)
    zcol = jnp.zeros((H + 2, 1, C), y.dtype)
    yl = jnp.concatenate([yp[:, 1:, :], zcol], axis=1)
    yr = jnp.concatenate([zcol, yp[:, : W - 1, :]], axis=1)
    srcs = [yr.reshape((H + 2) * W, C),
            yp.reshape((H + 2) * W, C),
            yl.reshape((H + 2) * W, C)]
    cols = jnp.concatenate(
        [srcs[j][i * W: i * W + H * W, :] for i in range(3) for j in range(3)],
        axis=-1)
    return jnp.dot(cols, w_ref[...], preferred_element_type=jnp.float32)


def _affine(s_ref, sq_ref, prm_ref, count):
    tot = jnp.sum(s_ref[...], axis=(0, 1))
    tot_sq = jnp.sum(sq_ref[...], axis=(0, 1))
    mean = tot / count
    var = jnp.maximum(tot_sq / count - mean * mean, 0.0)
    scale = prm_ref[0] / jnp.sqrt(var + 1e-5)
    shift = prm_ref[1] - mean * scale
    return scale, shift, prm_ref[2]


def _bn_prelu(x, aff):
    scale, shift, alpha = aff
    y = x * scale + shift
    return jnp.where(y >= 0.0, y, alpha * y)


def _stats_out(acc, s_ref, sq_ref):
    s_ref[0] = jnp.sum(acc, axis=0, keepdims=True)
    sq_ref[0] = jnp.sum(acc * acc, axis=0, keepdims=True)


def _convt_body(xd_ref, w_ref, o_ref, s_ref, sq_ref):
    xd = xd_ref[0]
    Hp, Wp, C = xd.shape
    y = xd[1:Hp - 1, 1:Wp - 1, :]
    acc = _conv3x3_flat(y, w_ref)
    o_ref[0] = acc.reshape(Hp - 2, Wp - 2, C)
    _stats_out(acc, s_ref, sq_ref)


def _make_conv0_body(count):
    def body(skc_ref, up_ref, s0_ref, sq0_ref, prm_ref, w_ref,
             o_ref, s_ref, sq_ref):
        u = _bn_prelu(up_ref[0], _affine(s0_ref, sq0_ref, prm_ref, count))
        y = jnp.concatenate([skc_ref[0], u], axis=-1)
        acc = _conv3x3_flat(y, w_ref)
        H, W, _ = y.shape
        o_ref[0] = acc.reshape(H, W, o_ref.shape[-1])
        _stats_out(acc, s_ref, sq_ref)
    return body


def _make_conv_body(count):
    def body(x_ref, s0_ref, sq0_ref, prm_ref, w_ref, o_ref, s_ref, sq_ref):
        y = _bn_prelu(x_ref[0], _affine(s0_ref, sq0_ref, prm_ref, count))
        acc = _conv3x3_flat(y, w_ref)
        H, W, _ = y.shape
        o_ref[0] = acc.reshape(H, W, o_ref.shape[-1])
        _stats_out(acc, s_ref, sq_ref)
    return body


def _make_resid_body(count):
    def body(y_ref, sy_ref, sqy_ref, prmy_ref, cb_ref, s2_ref, sq2_ref,
             prm2_ref, w_ref, b_ref, o_ref):
        y = _bn_prelu(y_ref[0], _affine(sy_ref, sqy_ref, prmy_ref, count))
        cb = _bn_prelu(cb_ref[0], _affine(s2_ref, sq2_ref, prm2_ref, count))
        H, W, C = y.shape
        n = w_ref.shape[1]
        out = jnp.dot(y.astype(jnp.bfloat16).reshape(H * W, C), w_ref[...],
                      preferred_element_type=jnp.float32)
        o_ref[0] = out.reshape(H, W, n) + b_ref[0] + cb
    return body


_PAR = pltpu.CompilerParams(dimension_semantics=("parallel",))


def _conv_stage(body, tensors, specs, out_shapes, out_specs, B):
    return pl.pallas_call(
        body,
        out_shape=out_shapes,
        grid=(B,),
        in_specs=specs,
        out_specs=out_specs,
        compiler_params=_PAR,
    )(*tensors)


def kernel(convT_w, convT_b, bn0_g, bn0_b, prelu0_a, conv0_w, conv0_b,
           bn1_g, bn1_b, prelu1_a, rb_c1_w, rb_c1_b, rb_bn1_g, rb_bn1_b,
           rb_prelu1_a, rb_c2_w, rb_c2_b, rb_bn2_g, rb_bn2_b, rb_prelu2_a,
           rb_c11_w, rb_c11_b, x, skip):
    xh = jnp.transpose(x, (0, 2, 3, 1)).astype(jnp.float32)
    skip = jnp.transpose(skip, (0, 2, 3, 1)).astype(jnp.float32)
    B, H, W, c = xh.shape
    Hs, Ws = skip.shape[1], skip.shape[2]
    n = conv0_w.shape[0]
    count = B * Hs * Ws

    op_h = Hs - ((H - 1) * 2 - 2 + 3)
    op_w = Ws - ((W - 1) * 2 - 2 + 3)
    hd, wd = 2 * H - 1, 2 * W - 1
    xd = jnp.zeros((B, hd, wd, c), jnp.float32).at[:, ::2, ::2, :].set(xh)
    xd = jnp.pad(xd, ((0, 0), (1, 1 + op_h), (1, 1 + op_w), (0, 0)))

    wt = jnp.transpose(jnp.flip(convT_w, (2, 3)), (2, 3, 0, 1)).astype(
        jnp.bfloat16).reshape(9 * c, c)
    w0 = jnp.transpose(conv0_w, (2, 3, 1, 0)).astype(jnp.bfloat16).reshape(
        18 * c, n)
    w1 = jnp.transpose(rb_c1_w, (2, 3, 1, 0)).astype(jnp.bfloat16).reshape(9 * n, n)
    w2 = jnp.transpose(rb_c2_w, (2, 3, 1, 0)).astype(jnp.bfloat16).reshape(9 * n, n)
    w11 = rb_c11_w[:, :, 0, 0].T.astype(jnp.bfloat16)
    b11 = rb_c11_b.reshape(1, -1).astype(jnp.float32)

    def prm(g, b_, alpha):
        a = jnp.broadcast_to(jnp.asarray(alpha, jnp.float32).reshape(()),
                             g.shape)
        return jnp.stack([g.astype(jnp.float32), b_.astype(jnp.float32), a], 0)

    prm0 = prm(bn0_g, bn0_b, prelu0_a)
    prm1 = prm(bn1_g, bn1_b, prelu1_a)
    prm_rb1 = prm(rb_bn1_g, rb_bn1_b, rb_prelu1_a)
    prm_rb2 = prm(rb_bn2_g, rb_bn2_b, rb_prelu2_a)

    full = lambda *dims: pl.BlockSpec(dims, lambda b: (0,) * len(dims))
    perb = lambda *dims: pl.BlockSpec((1,) + dims,
                                      lambda b: (b,) + (0,) * len(dims))
    stat_shapes = lambda ch: (jax.ShapeDtypeStruct((B, 1, ch), jnp.float32),
                              jax.ShapeDtypeStruct((B, 1, ch), jnp.float32))
    sspec = lambda ch: pl.BlockSpec((1, 1, ch), lambda b: (b, 0, 0))

    up, s0, sq0 = _conv_stage(
        _convt_body, (xd, wt),
        [perb(Hs + 2, Ws + 2, c), full(9 * c, c)],
        (jax.ShapeDtypeStruct((B, Hs, Ws, c), jnp.float32),) + stat_shapes(c),
        (perb(Hs, Ws, c), sspec(c), sspec(c)), B)

    y_raw, sy, sqy = _conv_stage(
        _make_conv0_body(count), (skip, up, s0, sq0, prm0, w0),
        [perb(Hs, Ws, c), perb(Hs, Ws, c), full(B, 1, c), full(B, 1, c),
         full(3, c), full(18 * c, n)],
        (jax.ShapeDtypeStruct((B, Hs, Ws, n), jnp.float32),) + stat_shapes(n),
        (perb(Hs, Ws, n), sspec(n), sspec(n)), B)

    cb1, s1, sq1 = _conv_stage(
        _make_conv_body(count), (y_raw, sy, sqy, prm1, w1),
        [perb(Hs, Ws, n), full(B, 1, n), full(B, 1, n), full(3, n),
         full(9 * n, n)],
        (jax.ShapeDtypeStruct((B, Hs, Ws, n), jnp.float32),) + stat_shapes(n),
        (perb(Hs, Ws, n), sspec(n), sspec(n)), B)

    cb2, s2, sq2 = _conv_stage(
        _make_conv_body(count), (cb1, s1, sq1, prm_rb1, w2),
        [perb(Hs, Ws, n), full(B, 1, n), full(B, 1, n), full(3, n),
         full(9 * n, n)],
        (jax.ShapeDtypeStruct((B, Hs, Ws, n), jnp.float32),) + stat_shapes(n),
        (perb(Hs, Ws, n), sspec(n), sspec(n)), B)

    out = _conv_stage(
        _make_resid_body(count),
        (y_raw, sy, sqy, prm1, cb2, s2, sq2, prm_rb2, w11, b11),
        [perb(Hs, Ws, n), full(B, 1, n), full(B, 1, n), full(3, n),
         perb(Hs, Ws, n), full(B, 1, n), full(B, 1, n), full(3, n),
         full(n, n), full(1, n)],
        jax.ShapeDtypeStruct((B, Hs, Ws, n), jnp.float32),
        perb(Hs, Ws, n), B)
    return jnp.transpose(out, (0, 3, 1, 2))


# speedup vs baseline: 1.3119x; 1.0418x over previous
import jax
import jax.numpy as jnp
from jax.experimental import pallas as pl
from jax.experimental.pallas import tpu as pltpu


def _conv3x3_flat(y, w_ref):
    y = y.astype(jnp.bfloat16)
    H, W, C = y.shape
    zrow = jnp.zeros((1, W, C), y.dtype)
    yp = jnp.concatenate([zrow, y, zrow], axis=0)
    zcol = jnp.zeros((H + 2, 1, C), y.dtype)
    yl = jnp.concatenate([yp[:, 1:, :], zcol], axis=1)
    yr = jnp.concatenate([zcol, yp[:, : W - 1, :]], axis=1)
    srcs = [yr.reshape((H + 2) * W, C),
            yp.reshape((H + 2) * W, C),
            yl.reshape((H + 2) * W, C)]
    cols = jnp.concatenate(
        [srcs[j][i * W: i * W + H * W, :] for i in range(3) for j in range(3)],
        axis=-1)
    return jnp.dot(cols, w_ref[...], preferred_element_type=jnp.float32)


def _affine(s_ref, sq_ref, prm_ref, count):
    tot = jnp.sum(s_ref[...], axis=(0, 1))
    tot_sq = jnp.sum(sq_ref[...], axis=(0, 1))
    mean = tot / count
    var = jnp.maximum(tot_sq / count - mean * mean, 0.0)
    scale = prm_ref[0] / jnp.sqrt(var + 1e-5)
    shift = prm_ref[1] - mean * scale
    return scale, shift, prm_ref[2]


def _bn_prelu(x, aff):
    scale, shift, alpha = aff
    y = x * scale + shift
    return jnp.where(y >= 0.0, y, alpha * y)


def _stats_out(acc, s_ref, sq_ref):
    s_ref[0] = jnp.sum(acc, axis=0, keepdims=True)
    sq_ref[0] = jnp.sum(acc * acc, axis=0, keepdims=True)


def _convt_body(xd_ref, w_ref, o_ref, s_ref, sq_ref):
    xd = xd_ref[0]
    Hp, Wp, C = xd.shape
    y = xd[1:Hp - 1, 1:Wp - 1, :]
    acc = _conv3x3_flat(y, w_ref)
    o_ref[0] = acc.reshape(Hp - 2, Wp - 2, C).astype(o_ref.dtype)
    _stats_out(acc, s_ref, sq_ref)


def _make_conv0_body(count):
    def body(skc_ref, up_ref, s0_ref, sq0_ref, prm_ref, w_ref,
             o_ref, s_ref, sq_ref):
        u = _bn_prelu(up_ref[0], _affine(s0_ref, sq0_ref, prm_ref, count))
        y = jnp.concatenate([skc_ref[0], u], axis=-1)
        acc = _conv3x3_flat(y, w_ref)
        H, W, _ = y.shape
        o_ref[0] = acc.reshape(H, W, o_ref.shape[-1]).astype(o_ref.dtype)
        _stats_out(acc, s_ref, sq_ref)
    return body


def _make_conv_body(count):
    def body(x_ref, s0_ref, sq0_ref, prm_ref, w_ref, o_ref, s_ref, sq_ref):
        y = _bn_prelu(x_ref[0], _affine(s0_ref, sq0_ref, prm_ref, count))
        acc = _conv3x3_flat(y, w_ref)
        H, W, _ = y.shape
        o_ref[0] = acc.reshape(H, W, o_ref.shape[-1]).astype(o_ref.dtype)
        _stats_out(acc, s_ref, sq_ref)
    return body


def _make_resid_body(count):
    def body(y_ref, sy_ref, sqy_ref, prmy_ref, cb_ref, s2_ref, sq2_ref,
             prm2_ref, w_ref, b_ref, o_ref):
        y = _bn_prelu(y_ref[0], _affine(sy_ref, sqy_ref, prmy_ref, count))
        cb = _bn_prelu(cb_ref[0], _affine(s2_ref, sq2_ref, prm2_ref, count))
        H, W, C = y.shape
        n = w_ref.shape[1]
        out = jnp.dot(y.astype(jnp.bfloat16).reshape(H * W, C), w_ref[...],
                      preferred_element_type=jnp.float32)
        o_ref[0] = out.reshape(H, W, n) + b_ref[0] + cb
    return body


_PAR = pltpu.CompilerParams(dimension_semantics=("parallel",))


def _conv_stage(body, tensors, specs, out_shapes, out_specs, B):
    return pl.pallas_call(
        body,
        out_shape=out_shapes,
        grid=(B,),
        in_specs=specs,
        out_specs=out_specs,
        compiler_params=_PAR,
    )(*tensors)


def kernel(convT_w, convT_b, bn0_g, bn0_b, prelu0_a, conv0_w, conv0_b,
           bn1_g, bn1_b, prelu1_a, rb_c1_w, rb_c1_b, rb_bn1_g, rb_bn1_b,
           rb_prelu1_a, rb_c2_w, rb_c2_b, rb_bn2_g, rb_bn2_b, rb_prelu2_a,
           rb_c11_w, rb_c11_b, x, skip):
    xh = jnp.transpose(x, (0, 2, 3, 1)).astype(jnp.float32)
    skip = jnp.transpose(skip, (0, 2, 3, 1)).astype(jnp.float32)
    B, H, W, c = xh.shape
    Hs, Ws = skip.shape[1], skip.shape[2]
    n = conv0_w.shape[0]
    count = B * Hs * Ws

    op_h = Hs - ((H - 1) * 2 - 2 + 3)
    op_w = Ws - ((W - 1) * 2 - 2 + 3)
    hd, wd = 2 * H - 1, 2 * W - 1
    xd = jnp.zeros((B, hd, wd, c), jnp.float32).at[:, ::2, ::2, :].set(xh)
    xd = jnp.pad(xd, ((0, 0), (1, 1 + op_h), (1, 1 + op_w), (0, 0)))

    wt = jnp.transpose(jnp.flip(convT_w, (2, 3)), (2, 3, 0, 1)).astype(
        jnp.bfloat16).reshape(9 * c, c)
    w0 = jnp.transpose(conv0_w, (2, 3, 1, 0)).astype(jnp.bfloat16).reshape(
        18 * c, n)
    w1 = jnp.transpose(rb_c1_w, (2, 3, 1, 0)).astype(jnp.bfloat16).reshape(9 * n, n)
    w2 = jnp.transpose(rb_c2_w, (2, 3, 1, 0)).astype(jnp.bfloat16).reshape(9 * n, n)
    w11 = rb_c11_w[:, :, 0, 0].T.astype(jnp.bfloat16)
    b11 = rb_c11_b.reshape(1, -1).astype(jnp.float32)

    def prm(g, b_, alpha):
        a = jnp.broadcast_to(jnp.asarray(alpha, jnp.float32).reshape(()),
                             g.shape)
        return jnp.stack([g.astype(jnp.float32), b_.astype(jnp.float32), a], 0)

    prm0 = prm(bn0_g, bn0_b, prelu0_a)
    prm1 = prm(bn1_g, bn1_b, prelu1_a)
    prm_rb1 = prm(rb_bn1_g, rb_bn1_b, rb_prelu1_a)
    prm_rb2 = prm(rb_bn2_g, rb_bn2_b, rb_prelu2_a)

    full = lambda *dims: pl.BlockSpec(dims, lambda b: (0,) * len(dims))
    perb = lambda *dims: pl.BlockSpec((1,) + dims,
                                      lambda b: (b,) + (0,) * len(dims))
    stat_shapes = lambda ch: (jax.ShapeDtypeStruct((B, 1, ch), jnp.float32),
                              jax.ShapeDtypeStruct((B, 1, ch), jnp.float32))
    sspec = lambda ch: pl.BlockSpec((1, 1, ch), lambda b: (b, 0, 0))

    up, s0, sq0 = _conv_stage(
        _convt_body, (xd, wt),
        [perb(Hs + 2, Ws + 2, c), full(9 * c, c)],
        (jax.ShapeDtypeStruct((B, Hs, Ws, c), jnp.bfloat16),) + stat_shapes(c),
        (perb(Hs, Ws, c), sspec(c), sspec(c)), B)

    y_raw, sy, sqy = _conv_stage(
        _make_conv0_body(count), (skip, up, s0, sq0, prm0, w0),
        [perb(Hs, Ws, c), perb(Hs, Ws, c), full(B, 1, c), full(B, 1, c),
         full(3, c), full(18 * c, n)],
        (jax.ShapeDtypeStruct((B, Hs, Ws, n), jnp.bfloat16),) + stat_shapes(n),
        (perb(Hs, Ws, n), sspec(n), sspec(n)), B)

    cb1, s1, sq1 = _conv_stage(
        _make_conv_body(count), (y_raw, sy, sqy, prm1, w1),
        [perb(Hs, Ws, n), full(B, 1, n), full(B, 1, n), full(3, n),
         full(9 * n, n)],
        (jax.ShapeDtypeStruct((B, Hs, Ws, n), jnp.bfloat16),) + stat_shapes(n),
        (perb(Hs, Ws, n), sspec(n), sspec(n)), B)

    cb2, s2, sq2 = _conv_stage(
        _make_conv_body(count), (cb1, s1, sq1, prm_rb1, w2),
        [perb(Hs, Ws, n), full(B, 1, n), full(B, 1, n), full(3, n),
         full(9 * n, n)],
        (jax.ShapeDtypeStruct((B, Hs, Ws, n), jnp.bfloat16),) + stat_shapes(n),
        (perb(Hs, Ws, n), sspec(n), sspec(n)), B)

    out = _conv_stage(
        _make_resid_body(count),
        (y_raw, sy, sqy, prm1, cb2, s2, sq2, prm_rb2, w11, b11),
        [perb(Hs, Ws, n), full(B, 1, n), full(B, 1, n), full(3, n),
         perb(Hs, Ws, n), full(B, 1, n), full(B, 1, n), full(3, n),
         full(n, n), full(1, n)],
        jax.ShapeDtypeStruct((B, Hs, Ws, n), jnp.float32),
        perb(Hs, Ws, n), B)
    return jnp.transpose(out, (0, 3, 1, 2))


# speedup vs baseline: 1.4599x; 1.1128x over previous
import jax
import jax.numpy as jnp
from jax.experimental import pallas as pl
from jax.experimental.pallas import tpu as pltpu


def _conv3x3_flat(y, w_ref):
    y = y.astype(jnp.bfloat16)
    H, W, C = y.shape
    zrow = jnp.zeros((1, W, C), y.dtype)
    yp = jnp.concatenate([zrow, y, zrow], axis=0)
    zcol = jnp.zeros((H + 2, 1, C), y.dtype)
    yl = jnp.concatenate([yp[:, 1:, :], zcol], axis=1)
    yr = jnp.concatenate([zcol, yp[:, : W - 1, :]], axis=1)
    srcs = [yr.reshape((H + 2) * W, C),
            yp.reshape((H + 2) * W, C),
            yl.reshape((H + 2) * W, C)]
    cols = jnp.concatenate(
        [srcs[j][i * W: i * W + H * W, :] for i in range(3) for j in range(3)],
        axis=-1)
    return jnp.dot(cols, w_ref[...], preferred_element_type=jnp.float32)


def _affine(s_ref, sq_ref, prm_ref, count):
    tot = jnp.sum(s_ref[...], axis=(0, 1))
    tot_sq = jnp.sum(sq_ref[...], axis=(0, 1))
    mean = tot / count
    var = jnp.maximum(tot_sq / count - mean * mean, 0.0)
    scale = prm_ref[0] / jnp.sqrt(var + 1e-5)
    shift = prm_ref[1] - mean * scale
    return scale, shift, prm_ref[2]


def _bn_prelu(x, aff):
    scale, shift, alpha = aff
    y = x * scale + shift
    return jnp.where(y >= 0.0, y, alpha * y)


def _stats_out(acc, s_ref, sq_ref):
    s_ref[0] = jnp.sum(acc, axis=0, keepdims=True)
    sq_ref[0] = jnp.sum(acc * acc, axis=0, keepdims=True)


def _convt_body(xd_ref, w_ref, o_ref, s_ref, sq_ref):
    xdu = xd_ref[0]
    Hd, Wd, C = xdu.shape
    yc = jnp.concatenate([xdu, jnp.zeros((Hd, 1, C), xdu.dtype)], axis=1)
    y = jnp.concatenate([yc, jnp.zeros((1, Wd + 1, C), xdu.dtype)], axis=0)
    acc = _conv3x3_flat(y, w_ref)
    o_ref[0] = acc.reshape(Hd + 1, Wd + 1, C).astype(o_ref.dtype)
    _stats_out(acc, s_ref, sq_ref)


def _make_conv0_body(count):
    def body(skc_ref, up_ref, s0_ref, sq0_ref, prm_ref, w_ref,
             o_ref, s_ref, sq_ref):
        u = _bn_prelu(up_ref[0], _affine(s0_ref, sq0_ref, prm_ref, count))
        y = jnp.concatenate([skc_ref[0], u], axis=-1)
        acc = _conv3x3_flat(y, w_ref)
        H, W, _ = y.shape
        o_ref[0] = acc.reshape(H, W, o_ref.shape[-1]).astype(o_ref.dtype)
        _stats_out(acc, s_ref, sq_ref)
    return body


def _make_conv_body(count):
    def body(x_ref, s0_ref, sq0_ref, prm_ref, w_ref, o_ref, s_ref, sq_ref):
        y = _bn_prelu(x_ref[0], _affine(s0_ref, sq0_ref, prm_ref, count))
        acc = _conv3x3_flat(y, w_ref)
        H, W, _ = y.shape
        o_ref[0] = acc.reshape(H, W, o_ref.shape[-1]).astype(o_ref.dtype)
        _stats_out(acc, s_ref, sq_ref)
    return body


def _make_resid_body(count):
    def body(y_ref, sy_ref, sqy_ref, prmy_ref, cb_ref, s2_ref, sq2_ref,
             prm2_ref, w_ref, b_ref, o_ref):
        y = _bn_prelu(y_ref[0], _affine(sy_ref, sqy_ref, prmy_ref, count))
        cb = _bn_prelu(cb_ref[0], _affine(s2_ref, sq2_ref, prm2_ref, count))
        H, W, C = y.shape
        n = w_ref.shape[1]
        out = jnp.dot(y.astype(jnp.bfloat16).reshape(H * W, C), w_ref[...],
                      preferred_element_type=jnp.float32)
        o_ref[0] = out.reshape(H, W, n) + b_ref[0] + cb
    return body


_PAR = pltpu.CompilerParams(dimension_semantics=("parallel",))


def _conv_stage(body, tensors, specs, out_shapes, out_specs, B):
    return pl.pallas_call(
        body,
        out_shape=out_shapes,
        grid=(B,),
        in_specs=specs,
        out_specs=out_specs,
        compiler_params=_PAR,
    )(*tensors)


def kernel(convT_w, convT_b, bn0_g, bn0_b, prelu0_a, conv0_w, conv0_b,
           bn1_g, bn1_b, prelu1_a, rb_c1_w, rb_c1_b, rb_bn1_g, rb_bn1_b,
           rb_prelu1_a, rb_c2_w, rb_c2_b, rb_bn2_g, rb_bn2_b, rb_prelu2_a,
           rb_c11_w, rb_c11_b, x, skip):
    xh = jnp.transpose(x, (0, 2, 3, 1)).astype(jnp.float32)
    skip = jnp.transpose(skip, (0, 2, 3, 1)).astype(jnp.float32)
    B, H, W, c = xh.shape
    Hs, Ws = skip.shape[1], skip.shape[2]
    n = conv0_w.shape[0]
    count = B * Hs * Ws

    op_h = Hs - ((H - 1) * 2 - 2 + 3)
    op_w = Ws - ((W - 1) * 2 - 2 + 3)
    hd, wd = 2 * H - 1, 2 * W - 1
    xd = jnp.zeros((B, hd, wd, c), jnp.float32).at[:, ::2, ::2, :].set(xh)
    del op_h, op_w

    wt = jnp.transpose(jnp.flip(convT_w, (2, 3)), (2, 3, 0, 1)).astype(
        jnp.bfloat16).reshape(9 * c, c)
    w0 = jnp.transpose(conv0_w, (2, 3, 1, 0)).astype(jnp.bfloat16).reshape(
        18 * c, n)
    w1 = jnp.transpose(rb_c1_w, (2, 3, 1, 0)).astype(jnp.bfloat16).reshape(9 * n, n)
    w2 = jnp.transpose(rb_c2_w, (2, 3, 1, 0)).astype(jnp.bfloat16).reshape(9 * n, n)
    w11 = rb_c11_w[:, :, 0, 0].T.astype(jnp.bfloat16)
    b11 = rb_c11_b.reshape(1, -1).astype(jnp.float32)

    def prm(g, b_, alpha):
        a = jnp.broadcast_to(jnp.asarray(alpha, jnp.float32).reshape(()),
                             g.shape)
        return jnp.stack([g.astype(jnp.float32), b_.astype(jnp.float32), a], 0)

    prm0 = prm(bn0_g, bn0_b, prelu0_a)
    prm1 = prm(bn1_g, bn1_b, prelu1_a)
    prm_rb1 = prm(rb_bn1_g, rb_bn1_b, rb_prelu1_a)
    prm_rb2 = prm(rb_bn2_g, rb_bn2_b, rb_prelu2_a)

    full = lambda *dims: pl.BlockSpec(dims, lambda b: (0,) * len(dims))
    perb = lambda *dims: pl.BlockSpec((1,) + dims,
                                      lambda b: (b,) + (0,) * len(dims))
    stat_shapes = lambda ch: (jax.ShapeDtypeStruct((B, 1, ch), jnp.float32),
                              jax.ShapeDtypeStruct((B, 1, ch), jnp.float32))
    sspec = lambda ch: pl.BlockSpec((1, 1, ch), lambda b: (b, 0, 0))

    up, s0, sq0 = _conv_stage(
        _convt_body, (xd, wt),
        [perb(hd, wd, c), full(9 * c, c)],
        (jax.ShapeDtypeStruct((B, Hs, Ws, c), jnp.bfloat16),) + stat_shapes(c),
        (perb(Hs, Ws, c), sspec(c), sspec(c)), B)

    y_raw, sy, sqy = _conv_stage(
        _make_conv0_body(count), (skip, up, s0, sq0, prm0, w0),
        [perb(Hs, Ws, c), perb(Hs, Ws, c), full(B, 1, c), full(B, 1, c),
         full(3, c), full(18 * c, n)],
        (jax.ShapeDtypeStruct((B, Hs, Ws, n), jnp.bfloat16),) + stat_shapes(n),
        (perb(Hs, Ws, n), sspec(n), sspec(n)), B)

    cb1, s1, sq1 = _conv_stage(
        _make_conv_body(count), (y_raw, sy, sqy, prm1, w1),
        [perb(Hs, Ws, n), full(B, 1, n), full(B, 1, n), full(3, n),
         full(9 * n, n)],
        (jax.ShapeDtypeStruct((B, Hs, Ws, n), jnp.bfloat16),) + stat_shapes(n),
        (perb(Hs, Ws, n), sspec(n), sspec(n)), B)

    cb2, s2, sq2 = _conv_stage(
        _make_conv_body(count), (cb1, s1, sq1, prm_rb1, w2),
        [perb(Hs, Ws, n), full(B, 1, n), full(B, 1, n), full(3, n),
         full(9 * n, n)],
        (jax.ShapeDtypeStruct((B, Hs, Ws, n), jnp.bfloat16),) + stat_shapes(n),
        (perb(Hs, Ws, n), sspec(n), sspec(n)), B)

    out = _conv_stage(
        _make_resid_body(count),
        (y_raw, sy, sqy, prm1, cb2, s2, sq2, prm_rb2, w11, b11),
        [perb(Hs, Ws, n), full(B, 1, n), full(B, 1, n), full(3, n),
         perb(Hs, Ws, n), full(B, 1, n), full(B, 1, n), full(3, n),
         full(n, n), full(1, n)],
        jax.ShapeDtypeStruct((B, Hs, Ws, n), jnp.float32),
        perb(Hs, Ws, n), B)
    return jnp.transpose(out, (0, 3, 1, 2))
